# baseline (speedup 1.0000x reference)
import functools

import jax
import jax.numpy as jnp
from jax import lax
from jax.experimental import pallas as pl
from jax.experimental.pallas import tpu as pltpu
from jax.experimental.pallas import tpu_sc as plsc

B = 1024
L = 50
BL = B * L
G = 10
D = 32

NC = 2
NS = 16
NW = NC * NS
LANES = 16

POS_PER_W = BL // NW
CHUNK = 160
NCHUNK = POS_PER_W // CHUNK
STREAM = 100
NGRP = CHUNK * G // STREAM
ROWS = CHUNK * G

_mesh = plsc.VectorSubcoreMesh(
    core_axis_name="c", subcore_axis_name="s", num_cores=NC, num_subcores=NS
)


@functools.partial(
    pl.kernel,
    out_type=jax.ShapeDtypeStruct((BL * G, D), jnp.float32),
    mesh=_mesh,
    compiler_params=pltpu.CompilerParams(
        use_tc_tiling_on_sc=False, needs_layout_passes=False),
    scratch_types=[
        pltpu.VMEM((NGRP, STREAM), jnp.int32),
        pltpu.VMEM((NGRP, STREAM), jnp.int32),
        pltpu.VMEM((CHUNK,), jnp.float32),
        pltpu.VMEM((CHUNK,), jnp.float32),
        pltpu.VMEM((ROWS, D), jnp.float32),
        pltpu.VMEM((ROWS, D), jnp.float32),
        pltpu.SemaphoreType.DMA,
        pltpu.SemaphoreType.DMA,
        pltpu.SemaphoreType.DMA,
        pltpu.SemaphoreType.DMA,
    ],
)
def _feat_embed(sel_hbm, mask_hbm, table_hbm, out_hbm,
                idx0, idx1, mask0, mask1, rows0, rows1,
                semg0, semg1, semo0, semo1):
    idx = (idx0, idx1)
    maskv = (mask0, mask1)
    rows = (rows0, rows1)
    semg = (semg0, semg1)
    semo = (semo0, semo1)

    wid = lax.axis_index("s") * NC + lax.axis_index("c")
    wpos0 = wid * POS_PER_W

    def stage_and_fire(c, b):
        pos0 = pl.multiple_of(wpos0 + c * CHUNK, 8)
        pltpu.sync_copy(sel_hbm.at[pl.ds(pos0 * G // STREAM, NGRP)], idx[b])
        pltpu.sync_copy(mask_hbm.at[pl.ds(pos0, CHUNK)], maskv[b])
        for g in range(NGRP):
            pltpu.async_copy(
                table_hbm.at[idx[b].at[g]],
                rows[b].at[pl.ds(g * STREAM, STREAM)],
                semg[b])

    def wait_gathers(b):
        for g in range(NGRP):
            pltpu.make_async_copy(
                table_hbm.at[idx[b].at[g]],
                rows[b].at[pl.ds(g * STREAM, STREAM)],
                semg[b]).wait()

    def out_slice(c):
        row0 = pl.multiple_of((wpos0 + c * CHUNK) * G, 8)
        return out_hbm.at[pl.ds(row0, ROWS)]

    def drain_out(c, b):
        pltpu.make_async_copy(rows[b], out_slice(c), semo[b]).wait()

    stage_and_fire(0, 0)

    def pair_body(cc, carry):
        for b in (0, 1):
            c = cc * 2 + b

            @pl.when(c + 1 < NCHUNK)
            def _fire_next():
                @pl.when(c >= 1)
                def _drain_prev():
                    drain_out(c - 1, 1 - b)
                stage_and_fire(c + 1, 1 - b)

            wait_gathers(b)

            def pos_body(p, carry2):
                m = plsc.load_gather(maskv[b], [lax.broadcast(p, (LANES,))])
                for r in range(G):
                    row = p * G + r
                    for h in (0, LANES):
                        rows[b][row, pl.ds(h, LANES)] = (
                            rows[b][row, pl.ds(h, LANES)] * m)
                return carry2
            lax.fori_loop(0, CHUNK, pos_body, 0, unroll=2)

            pltpu.async_copy(rows[b], out_slice(c), semo[b])
        return carry

    lax.fori_loop(0, NCHUNK // 2, pair_body, 0, unroll=False)
    drain_out(NCHUNK - 2, 0)
    drain_out(NCHUNK - 1, 1)


def kernel(feat_matrix, padding, embed_table):
    sel = feat_matrix[:, :, :G].reshape(BL * G // STREAM, STREAM)
    sel = sel.astype(jnp.int32)
    maskf = 1.0 - padding.reshape(-1).astype(jnp.float32)
    out = _feat_embed(sel, maskf, embed_table)
    return out.reshape(B, L, G * D)
